# Initial kernel scaffold; baseline (speedup 1.0000x reference)
import jax
import jax.numpy as jnp
from jax.experimental import pallas as pl


def kernel(x, edge_index, batch, W1, a_src1, a_dst1, b1, W2, a_src2, a_dst2, b2, Wo, bo):
    raise NotImplementedError("write your pallas kernel here")


# speedup vs baseline: 1.0508x; 1.0508x over previous
import functools

import jax
import jax.numpy as jnp
from jax.experimental import pallas as pl


def _final_matmul_kernel(pooled_ref, wo_ref, bo_ref, out_ref):
    out_ref[...] = (
        jnp.dot(pooled_ref[...], wo_ref[...], preferred_element_type=jnp.float32)
        + bo_ref[...]
    )


def _final_matmul(pooled, Wo, bo):
    G = pooled.shape[0]
    return pl.pallas_call(
        _final_matmul_kernel,
        out_shape=jax.ShapeDtypeStruct((G, Wo.shape[1]), jnp.float32),
    )(pooled, Wo, bo.reshape(1, -1))


def _gat_layer_xla(x, src, dst, W, a_src, a_dst, bias, heads, out_ch, concat):
    N = x.shape[0]
    h = (x @ W).reshape(N, heads, out_ch)
    alpha_src = (h * a_src).sum(-1)
    alpha_dst = (h * a_dst).sum(-1)
    e = alpha_src[src] + alpha_dst[dst]
    w = jnp.exp(jnp.maximum(e, 0.2 * e))
    denom = jax.ops.segment_sum(w, dst, num_segments=N)
    msg = jax.ops.segment_sum(w[:, :, None] * h[src], dst, num_segments=N)
    e_self = alpha_src + alpha_dst
    w_self = jnp.exp(jnp.maximum(e_self, 0.2 * e_self))
    out = (msg + w_self[:, :, None] * h) / (denom + w_self + 1e-16)[:, :, None]
    if concat:
        out = out.reshape(N, heads * out_ch)
    else:
        out = out.mean(axis=1)
    return out + bias


def kernel(x, edge_index, batch, W1, a_src1, a_dst1, b1, W2, a_src2, a_dst2, b2, Wo, bo):
    src = edge_index[0]
    dst = edge_index[1]
    h = jax.nn.relu(_gat_layer_xla(x, src, dst, W1, a_src1, a_dst1, b1, 4, 32, True))
    h = jax.nn.relu(_gat_layer_xla(h, src, dst, W2, a_src2, a_dst2, b2, 1, 64, False))
    num_graphs = 256
    sums = jax.ops.segment_sum(h, batch, num_segments=num_graphs)
    cnt = jax.ops.segment_sum(jnp.ones((h.shape[0], 1), h.dtype), batch, num_segments=num_graphs)
    pooled = sums / jnp.maximum(cnt, 1.0)
    return _final_matmul(pooled, Wo, bo)


# speedup vs baseline: 41.2523x; 39.2570x over previous
import functools

import jax
import jax.numpy as jnp
import numpy as np
from jax import lax
from jax.experimental import pallas as pl
from jax.experimental.pallas import tpu as pltpu
from jax.experimental.pallas import tpu_sc as plsc

N_NODES = 100000
N_EDGES = 3200000
NUM_GRAPHS = 256

NC, NS, L = 2, 16, 16
CH = 800
SUB = 80
NSUB = CH // SUB
EPT_ACC = N_EDGES // NS
EPT_ATTN = N_EDGES // (NC * NS)
NPT = N_NODES // NS
ZCH = 625

BN = 400
GRID_N = N_NODES // BN

NPAD = 101376
PCH = 192
PACC = 1408


def _prep1_body(x_ref, w_ref, as_ref, ad_ref, h_ref, asT_ref, adT_ref,
                wself_ref):
    i = pl.program_id(0)
    h = jnp.dot(x_ref[...], w_ref[...], preferred_element_type=jnp.float32)
    h_ref[...] = h
    hh = h.reshape(BN, 4, 32)
    a_s = (hh * as_ref[...][None]).sum(-1)
    a_d = (hh * ad_ref[...][None]).sum(-1)
    asT_ref[:, i, :] = a_s.T
    adT_ref[:, i, :] = a_d.T
    e = a_s + a_d
    wself_ref[...] = jnp.exp(jnp.maximum(e, 0.2 * e))


def _prep1(x, W1, a_src1, a_dst1):
    return pl.pallas_call(
        _prep1_body,
        grid=(GRID_N,),
        in_specs=[
            pl.BlockSpec((BN, 6), lambda i: (i, 0)),
            pl.BlockSpec((6, 128), lambda i: (0, 0)),
            pl.BlockSpec((4, 32), lambda i: (0, 0)),
            pl.BlockSpec((4, 32), lambda i: (0, 0)),
        ],
        out_specs=[
            pl.BlockSpec((BN, 128), lambda i: (i, 0)),
            pl.BlockSpec((4, GRID_N, BN), lambda i: (0, 0, 0)),
            pl.BlockSpec((4, GRID_N, BN), lambda i: (0, 0, 0)),
            pl.BlockSpec((BN, 4), lambda i: (i, 0)),
        ],
        out_shape=[
            jax.ShapeDtypeStruct((N_NODES, 128), jnp.float32),
            jax.ShapeDtypeStruct((4, GRID_N, BN), jnp.float32),
            jax.ShapeDtypeStruct((4, GRID_N, BN), jnp.float32),
            jax.ShapeDtypeStruct((N_NODES, 4), jnp.float32),
        ],
    )(x, W1, a_src1, a_dst1)


def _combine1_body(*refs):
    msg_refs = refs[:8]
    den_refs = refs[8:12]
    h_ref, ws_ref, b_ref, w2_ref, as2_ref, ad2_ref = refs[12:18]
    h2_ref, as2T_ref, ad2T_ref, ws2_ref = refs[18:]
    i = pl.program_id(0)
    msg = jnp.concatenate([m[...] for m in msg_refs], axis=1)
    msg = msg.reshape(BN, 4, 32)
    den = jnp.concatenate([d[...][:, 0:1] for d in den_refs], axis=1)
    ws = ws_ref[...]
    h = h_ref[...].reshape(BN, 4, 32)
    num = msg + ws[:, :, None] * h
    out = num / (den + ws + 1e-16)[:, :, None]
    hin = jnp.maximum(out.reshape(BN, 128) + b_ref[...], 0.0)
    h2 = jnp.dot(hin, w2_ref[...], preferred_element_type=jnp.float32)
    h2_ref[...] = h2
    s2 = (h2 * as2_ref[...]).sum(-1, keepdims=True)
    d2 = (h2 * ad2_ref[...]).sum(-1, keepdims=True)
    as2T_ref[:, i, :] = s2.T
    ad2T_ref[:, i, :] = d2.T
    e2 = s2 + d2
    ws2_ref[...] = jnp.exp(jnp.maximum(e2, 0.2 * e2))


def _combine1(msgs, dens, h1, wself1, b1, W2, a_src2, a_dst2):
    return pl.pallas_call(
        _combine1_body,
        grid=(GRID_N,),
        in_specs=(
            [pl.BlockSpec((BN, 16), lambda i: (i, 0)) for _ in range(12)]
            + [
                pl.BlockSpec((BN, 128), lambda i: (i, 0)),
                pl.BlockSpec((BN, 4), lambda i: (i, 0)),
                pl.BlockSpec((1, 128), lambda i: (0, 0)),
                pl.BlockSpec((128, 64), lambda i: (0, 0)),
                pl.BlockSpec((1, 64), lambda i: (0, 0)),
                pl.BlockSpec((1, 64), lambda i: (0, 0)),
            ]
        ),
        out_specs=[
            pl.BlockSpec((BN, 64), lambda i: (i, 0)),
            pl.BlockSpec((1, GRID_N, BN), lambda i: (0, 0, 0)),
            pl.BlockSpec((1, GRID_N, BN), lambda i: (0, 0, 0)),
            pl.BlockSpec((BN, 1), lambda i: (i, 0)),
        ],
        out_shape=[
            jax.ShapeDtypeStruct((N_NODES, 64), jnp.float32),
            jax.ShapeDtypeStruct((1, GRID_N, BN), jnp.float32),
            jax.ShapeDtypeStruct((1, GRID_N, BN), jnp.float32),
            jax.ShapeDtypeStruct((N_NODES, 1), jnp.float32),
        ],
    )(*msgs, *dens, h1, wself1, b1, W2, a_src2, a_dst2)


def _combine2_body(*refs):
    msg_refs = refs[:4]
    den_ref, h_ref, ws_ref, b_ref = refs[4:8]
    out_refs = refs[8:]
    msg = jnp.concatenate([m[...] for m in msg_refs], axis=1)
    den = den_ref[...][:, 0:1]
    ws = ws_ref[...]
    num = msg + ws * h_ref[...]
    out = num / (den + ws + 1e-16)
    out = jnp.maximum(out + b_ref[...], 0.0)
    for r in range(4):
        out_refs[r][...] = out[:, 16 * r:16 * r + 16]


def _combine2(msgs, den2, h2, wself2, b2):
    return pl.pallas_call(
        _combine2_body,
        grid=(GRID_N,),
        in_specs=(
            [pl.BlockSpec((BN, 16), lambda i: (i, 0)) for _ in range(5)]
            + [
                pl.BlockSpec((BN, 64), lambda i: (i, 0)),
                pl.BlockSpec((BN, 1), lambda i: (i, 0)),
                pl.BlockSpec((1, 64), lambda i: (0, 0)),
            ]
        ),
        out_specs=[pl.BlockSpec((BN, 16), lambda i: (i, 0))
                   for _ in range(4)],
        out_shape=[jax.ShapeDtypeStruct((N_NODES, 16), jnp.float32)
                   for _ in range(4)],
    )(*msgs, den2, h2, wself2, b2)


def _pool_finish_body(p_ref, wo_ref, bo_ref, out_ref):
    p = p_ref[...]
    sums = p[:1040].reshape(4, 260, 16)[:, :256, :]
    sums = jnp.transpose(sums, (1, 0, 2)).reshape(256, 64)
    cnt = p[1040:1300].reshape(260, 16)[:256, 0:1]
    pooled = sums / jnp.maximum(cnt, 1.0)
    out_ref[...] = (
        jnp.dot(pooled, wo_ref[...], preferred_element_type=jnp.float32)
        + bo_ref[...]
    )


def _pool_finish(pool8, Wo, bo):
    return pl.pallas_call(
        _pool_finish_body,
        out_shape=jax.ShapeDtypeStruct((NUM_GRAPHS, 32), jnp.float32),
    )(pool8, Wo, bo.reshape(1, -1))


_MESH = plsc.VectorSubcoreMesh(core_axis_name="c", subcore_axis_name="s",
                               num_cores=NC, num_subcores=NS)


def _attn_kernel(H):
    @functools.partial(
        pl.kernel,
        out_type=jax.ShapeDtypeStruct((H * N_EDGES,), jnp.float32),
        mesh=_MESH,
        scratch_types=[
            pltpu.VMEM((CH,), jnp.int32),
            pltpu.VMEM((NSUB, SUB), jnp.int32),
            pltpu.VMEM((CH,), jnp.float32),
            pltpu.VMEM((CH,), jnp.float32),
            pltpu.SemaphoreType.DMA,
        ],
    )
    def k(asT, adT, src, dst, w_out, iv, gix, ev, av, sem):
        c = lax.axis_index("c")
        s = lax.axis_index("s")
        wid = c * NS + s

        def gather_alpha(tab, endpoints, eb, h, dstv):
            pltpu.sync_copy(endpoints.at[pl.ds(eb, CH)], iv)
            for m in range(CH // L):
                r, col = m // (SUB // L), L * (m % (SUB // L))
                gix[r, pl.ds(col, L)] = iv[pl.ds(m * L, L)] + h * N_NODES
            descs = [
                pltpu.async_copy(tab.at[gix.at[sb]],
                                 dstv.at[pl.ds(sb * SUB, SUB)], sem)
                for sb in range(NSUB)
            ]
            for d in descs:
                d.wait()

        for h in range(H):

            def body(ci, _):
                eb = wid * EPT_ATTN + ci * CH
                gather_alpha(asT, src, eb, h, ev)
                gather_alpha(adT, dst, eb, h, av)

                def g(kk, _):
                    e = ev[pl.ds(kk * L, L)] + av[pl.ds(kk * L, L)]
                    e = jnp.maximum(e, 0.2 * e)
                    ev[pl.ds(kk * L, L)] = jnp.exp(e)
                    return 0

                lax.fori_loop(0, CH // L, g, 0)
                pltpu.sync_copy(ev, w_out.at[pl.ds(h * N_EDGES + eb, CH)])
                return 0

            lax.fori_loop(0, EPT_ATTN // CH, body, 0)

    return k


RB_FULL = 6256
RB_LAST = N_NODES - (NS - 1) * RB_FULL


def _acc_kernel(G, NDEN, mhpg):
    n_groups = G + NDEN

    @functools.partial(
        pl.kernel,
        out_type=[jax.ShapeDtypeStruct((N_NODES, L), jnp.float32)
                  for _ in range(n_groups)],
        mesh=_MESH,
        compiler_params=pltpu.CompilerParams(use_tc_tiling_on_sc=False),
        scratch_types=[
            pltpu.VMEM_SHARED((N_NODES, L), jnp.float32),
            pltpu.VMEM((CH,), jnp.int32),
            pltpu.VMEM((CH,), jnp.int32),
            pltpu.VMEM((CH,), jnp.float32),
            pltpu.VMEM((CH, L), jnp.float32),
            pltpu.VMEM((NSUB, SUB), jnp.int32),
            pltpu.VMEM((NSUB, SUB), jnp.int32),
            pltpu.VMEM((ZCH, L), jnp.float32),
            pltpu.SemaphoreType.DMA,
        ],
    )
    def k(h8, w, src, dst, *refs):
        outs = refs[:n_groups]
        accS, sv, dv, wv, rows, gix, six, zb, sem = refs[n_groups:]
        c = lax.axis_index("c")
        s = lax.axis_index("s")

        def zfill(i, _):
            zb[i] = jnp.zeros((L,), jnp.float32)
            return 0

        lax.fori_loop(0, ZCH, zfill, 0)

        def zero_acc():
            for z in range(NPT // ZCH):
                pltpu.sync_copy(zb, accS.at[pl.ds(s * NPT + z * ZCH, ZCH)])

        def load_edges(ci, head, need_src):
            eb = s * EPT_ACC + ci * CH
            if need_src:
                pltpu.sync_copy(src.at[pl.ds(eb, CH)], sv)
            pltpu.sync_copy(dst.at[pl.ds(eb, CH)], dv)
            pltpu.sync_copy(w.at[pl.ds(head * N_EDGES + eb, CH)], wv)

        def build_scatter_idx():
            for m in range(CH // L):
                r, col = m // (SUB // L), L * (m % (SUB // L))
                six[r, pl.ds(col, L)] = dv[pl.ds(m * L, L)]

        def do_scatter():
            for sb in range(NSUB):
                pltpu.sync_copy(rows.at[pl.ds(sb * SUB, SUB)],
                                accS.at[six.at[sb]], add=True)

        def readback(out_ref):
            @pl.when(s < NS - 1)
            def _():
                pltpu.sync_copy(accS.at[pl.ds(s * RB_FULL, RB_FULL)],
                                out_ref.at[pl.ds(s * RB_FULL, RB_FULL)])

            @pl.when(s == NS - 1)
            def _():
                pltpu.sync_copy(
                    accS.at[pl.ds((NS - 1) * RB_FULL, RB_LAST)],
                    out_ref.at[pl.ds((NS - 1) * RB_FULL, RB_LAST)])

        def msg_pass(g):
            head = g // mhpg

            def mchunk(ci, _):
                load_edges(ci, head, True)
                for m in range(CH // L):
                    r, col = m // (SUB // L), L * (m % (SUB // L))
                    gix[r, pl.ds(col, L)] = sv[pl.ds(m * L, L)] * G + g
                build_scatter_idx()
                descs = [
                    pltpu.async_copy(h8.at[gix.at[sb]],
                                     rows.at[pl.ds(sb * SUB, SUB)], sem)
                    for sb in range(NSUB)
                ]
                for d in descs:
                    d.wait()

                def scale(kk, _):
                    wvv = wv[pl.ds(kk * L, L)]
                    for j in range(L):
                        i = kk * L + j
                        rows[i] = rows[i] * jnp.full((L,), wvv[j],
                                                     jnp.float32)
                    return 0

                lax.fori_loop(0, CH // L, scale, 0)
                do_scatter()
                return 0

            def run():
                lax.fori_loop(0, EPT_ACC // CH, mchunk, 0)

            return run

        def den_pass(head):
            def dchunk(ci, _):
                load_edges(ci, head, False)
                build_scatter_idx()

                def bcast(kk, _):
                    wvv = wv[pl.ds(kk * L, L)]
                    for j in range(L):
                        rows[kk * L + j] = jnp.full((L,), wvv[j],
                                                    jnp.float32)
                    return 0

                lax.fori_loop(0, CH // L, bcast, 0)
                do_scatter()
                return 0

            def run():
                lax.fori_loop(0, EPT_ACC // CH, dchunk, 0)

            return run

        sched = []
        for g in range(G):
            sched.append((g % NC, g, msg_pass(g)))
        for d_ in range(NDEN):
            sched.append((d_ % NC, G + d_, den_pass(d_)))

        for cc, oidx, run in sched:
            pl.when(c == cc)(zero_acc)
            plsc.subcore_barrier()
            pl.when(c == cc)(run)
            plsc.subcore_barrier()
            pl.when(c == cc)(lambda oidx=oidx: readback(outs[oidx]))
            plsc.subcore_barrier()

    return k


@functools.partial(
    pl.kernel,
    out_type=jax.ShapeDtypeStruct((PACC, L), jnp.float32),
    mesh=_MESH,
    compiler_params=pltpu.CompilerParams(use_tc_tiling_on_sc=False),
    scratch_types=[
        pltpu.VMEM_SHARED((PACC, L), jnp.float32),
        pltpu.VMEM((PCH,), jnp.int32),
        pltpu.VMEM((PCH, L), jnp.float32),
        pltpu.VMEM((2, 96), jnp.int32),
        pltpu.VMEM((PCH, L), jnp.float32),
        pltpu.VMEM((PACC // NS, L), jnp.float32),
    ],
)
def _pool_kernel(hq0, hq1, hq2, hq3, batchp, pool8, accS, bv, rows, six,
                 ones, zb):
    c = lax.axis_index("c")
    s = lax.axis_index("s")
    zrows = PACC // NS
    hqs = (hq0, hq1, hq2, hq3)

    def zfill(i, _):
        zb[i] = jnp.zeros((L,), jnp.float32)
        return 0

    lax.fori_loop(0, zrows, zfill, 0)

    def ofill(i, _):
        ones[i] = jnp.full((L,), 1.0, jnp.float32)
        return 0

    lax.fori_loop(0, PCH, ofill, 0)
    pltpu.sync_copy(zb, accS.at[pl.ds(s * zrows, zrows)])
    plsc.subcore_barrier()

    @pl.when(c == 0)
    def _work():
        def chunk(ci, _):
            nb = s * (NPAD // NS) + ci * PCH
            pltpu.sync_copy(batchp.at[pl.ds(nb, PCH)], bv)
            for r in range(4):
                pltpu.sync_copy(hqs[r].at[pl.ds(nb, PCH)], rows)
                for t in range(PCH // L):
                    b16 = bv[pl.ds(L * t, L)]
                    six[t // 6, pl.ds(L * (t % 6), L)] = b16 + 260 * r
                for sb in range(2):
                    pltpu.sync_copy(rows.at[pl.ds(96 * sb, 96)],
                                    accS.at[six.at[sb]], add=True)
            for t in range(PCH // L):
                b16 = bv[pl.ds(L * t, L)]
                six[t // 6, pl.ds(L * (t % 6), L)] = b16 + 1040
            for sb in range(2):
                pltpu.sync_copy(ones.at[pl.ds(96 * sb, 96)],
                                accS.at[six.at[sb]], add=True)
            return 0

        lax.fori_loop(0, NPAD // NS // PCH, chunk, 0)

    plsc.subcore_barrier()

    @pl.when(c == 0)
    def _rb():
        pltpu.sync_copy(accS.at[pl.ds(s * zrows, zrows)],
                        pool8.at[pl.ds(s * zrows, zrows)])


def kernel(x, edge_index, batch, W1, a_src1, a_dst1, b1, W2, a_src2, a_dst2,
           b2, Wo, bo):
    src = edge_index[0]
    dst = edge_index[1]

    h1, asT1, adT1, wself1 = _prep1(x, W1, a_src1.reshape(4, 32),
                                    a_dst1.reshape(4, 32))

    w1 = _attn_kernel(4)(asT1.reshape(-1), adT1.reshape(-1), src, dst)
    acc1 = _acc_kernel(8, 4, 2)(h1.reshape(N_NODES * 8, L), w1, src, dst)

    h2, asT2, adT2, wself2 = _combine1(acc1[:8], acc1[8:], h1, wself1,
                                       b1.reshape(1, -1), W2,
                                       a_src2.reshape(1, 64),
                                       a_dst2.reshape(1, 64))

    w2 = _attn_kernel(1)(asT2.reshape(-1), adT2.reshape(-1), src, dst)
    acc2 = _acc_kernel(4, 1, 4)(h2.reshape(N_NODES * 4, L), w2, src, dst)

    hqs = _combine2(acc2[:4], acc2[4], h2, wself2, b2.reshape(1, -1))

    hqp = [jnp.pad(hq, ((0, NPAD - N_NODES), (0, 0))) for hq in hqs]
    batchp = jnp.pad(batch, (0, NPAD - N_NODES), constant_values=NUM_GRAPHS)
    pool8 = _pool_kernel(*hqp, batchp)

    return _pool_finish(pool8, Wo, bo)
